# baseline (speedup 1.0000x reference)
import functools

import jax
import jax.numpy as jnp
from jax import lax
from jax.experimental import pallas as pl
from jax.experimental.pallas import tpu as pltpu
from jax.experimental.pallas import tpu_sc as plsc

B = 1024
V = 100000
D = 32

NC = 2
NS = 16
NW = NC * NS
B_PER_W = B // NW

VSPLIT = 50176


BB = 128
VB = 12544
SB = VB // 128
NSKIP = VSPLIT // VB
NVB = (V + VB - 1) // VB
NJ_TC = NVB - NSKIP
NR = BB // 8


def _tc_body(x_ref, idx_ref, max_ref, m_ref, s_ref):
    j = pl.program_id(1)

    @pl.when(j == 0)
    def _():
        m_ref[...] = jnp.full((BB, 128), -jnp.inf, jnp.float32)
        s_ref[...] = jnp.zeros((BB, 128), jnp.int32)

    def scan_block(last):
        m = [m_ref[r * 8:(r + 1) * 8, :] for r in range(NR)]
        s = [s_ref[r * 8:(r + 1) * 8, :] for r in range(NR)]
        tail = V - (NVB - 1) * VB
        lane = lax.broadcasted_iota(jnp.int32, (8, 128), 1)
        for k in range(SB):
            base = k * 128
            if last and base >= tail:
                break
            masked = last and base + 128 > tail
            for r in range(NR):
                v = x_ref[r * 8:(r + 1) * 8, base:base + 128]
                if masked:
                    v = jnp.where(lane < (tail - base), v, -jnp.inf)
                gk = ((NVB - 1) * SB + k) if last else ((j + NSKIP) * SB + k)
                cmp = v > m[r]
                m[r] = jnp.where(cmp, v, m[r])
                s[r] = jnp.where(cmp, gk, s[r])
        return m, s, lane

    @pl.when(j < NJ_TC - 1)
    def _():
        m, s, _ = scan_block(last=False)
        for r in range(NR):
            m_ref[r * 8:(r + 1) * 8, :] = m[r]
            s_ref[r * 8:(r + 1) * 8, :] = s[r]

    @pl.when(j == NJ_TC - 1)
    def _():
        m, s, lane = scan_block(last=True)
        for r in range(NR):
            rowmax = jnp.max(m[r], axis=1, keepdims=True)
            col = s[r] * 128 + lane
            idx_ref[r * 8:(r + 1) * 8, :] = jnp.min(
                jnp.where(m[r] == rowmax, col, jnp.int32(2**30)),
                axis=1, keepdims=True,
            )
            max_ref[r * 8:(r + 1) * 8, :] = rowmax


_tc_call = pl.pallas_call(
    _tc_body,
    grid=(B // BB, NJ_TC),
    in_specs=[pl.BlockSpec((BB, VB), lambda i, j: (i, j + NSKIP))],
    out_specs=[
        pl.BlockSpec((BB, 1), lambda i, j: (i, 0)),
        pl.BlockSpec((BB, 1), lambda i, j: (i, 0)),
    ],
    out_shape=[
        jax.ShapeDtypeStruct((B, 1), jnp.int32),
        jax.ShapeDtypeStruct((B, 1), jnp.float32),
    ],
    scratch_shapes=[
        pltpu.VMEM((BB, 128), jnp.float32),
        pltpu.VMEM((BB, 128), jnp.int32),
    ],
)


CW = 3584
NCH = VSPLIT // CW
GPC = CW // 16
HALF = 16


@functools.lru_cache(maxsize=1)
def _make_sc_argmax():
    @functools.partial(
        pl.kernel,
        out_type=[
            jax.ShapeDtypeStruct((B * 16,), jnp.float32),
            jax.ShapeDtypeStruct((B * 16,), jnp.int32),
        ],
        mesh=plsc.VectorSubcoreMesh(
            core_axis_name="c", subcore_axis_name="s", num_cores=NC,
            num_subcores=NS,
        ),
        scratch_types=[
            pltpu.VMEM((HALF * CW,), jnp.float32),
            pltpu.VMEM((HALF * CW,), jnp.float32),
            pltpu.VMEM((B_PER_W * 16,), jnp.float32),
            pltpu.VMEM((B_PER_W * 16,), jnp.int32),
            pltpu.SemaphoreType.DMA,
            pltpu.SemaphoreType.DMA,
        ],
    )
    def _sc_argmax(x_hbm, pm_hbm, ps_hbm, buf0, buf1, pm_v, ps_v,
                   sem0, sem1):
        wid = lax.axis_index("s") * NC + lax.axis_index("c")
        rbase = wid * B_PER_W
        bufs = (buf0, buf1)
        sems = (sem0, sem1)

        for hh in range(2):
            def mkcopies(c, b):
                c0 = pl.multiple_of(c * CW, 128)
                return [
                    pltpu.make_async_copy(
                        x_hbm.at[rbase + hh * HALF + r, pl.ds(c0, CW)],
                        bufs[b].at[pl.ds(r * CW, CW)],
                        sems[b],
                    )
                    for r in range(HALF)
                ]

            for cp in mkcopies(0, 0):
                cp.start()
            for cp in mkcopies(1, 1):
                cp.start()

            def outer(o, carry):
                st = carry
                for b in range(2):
                    c = 2 * o + b
                    for cp in mkcopies(c, b):
                        cp.wait()

                    def gloop(g, st_):
                        ms, ss = st_
                        gcol = jnp.full((16,), c * GPC + g, jnp.int32)
                        nm, ns_ = [], []
                        for r in range(HALF):
                            v = bufs[b][pl.ds(
                                pl.multiple_of(r * CW, 16) + g * 16, 16)]
                            cmp = v > ms[r]
                            nm.append(jnp.where(cmp, v, ms[r]))
                            ns_.append(jnp.where(cmp, gcol, ss[r]))
                        return (tuple(nm), tuple(ns_))

                    st = lax.fori_loop(0, GPC, gloop, st)

                    @pl.when(c + 2 < NCH)
                    def _():
                        for cp in mkcopies(c + 2, b):
                            cp.start()
                return st

            init = (
                tuple(jnp.full((16,), -jnp.inf, jnp.float32)
                      for _ in range(HALF)),
                tuple(jnp.zeros((16,), jnp.int32) for _ in range(HALF)),
            )
            ms, ss = lax.fori_loop(0, NCH // 2, outer, init)

            for r in range(HALF):
                base16 = (hh * HALF + r) * 16
                pm_v[pl.ds(base16, 16)] = ms[r]
                ps_v[pl.ds(base16, 16)] = ss[r]
        pltpu.sync_copy(pm_v, pm_hbm.at[pl.ds(rbase * 16, B_PER_W * 16)])
        pltpu.sync_copy(ps_v, ps_hbm.at[pl.ds(rbase * 16, B_PER_W * 16)])

    return _sc_argmax


@functools.lru_cache(maxsize=1)
def _make_sc_gather():
    @functools.partial(
        pl.kernel,
        out_type=jax.ShapeDtypeStruct((B, D), jnp.float32),
        mesh=plsc.VectorSubcoreMesh(
            core_axis_name="c", subcore_axis_name="s", num_cores=NC,
            num_subcores=NS,
        ),
        scratch_types=[
            pltpu.VMEM((B_PER_W,), jnp.int32),
            pltpu.VMEM((B_PER_W, D), jnp.float32),
            pltpu.SemaphoreType.DMA,
        ],
        compiler_params=pltpu.CompilerParams(use_tc_tiling_on_sc=False),
    )
    def _sc_gather(table_hbm, idx_hbm, out_hbm, idx_v, rows_v, sem):
        wid = lax.axis_index("s") * NC + lax.axis_index("c")
        base = wid * B_PER_W
        pltpu.sync_copy(idx_hbm.at[pl.ds(base, B_PER_W)], idx_v)
        pltpu.async_copy(table_hbm.at[idx_v], rows_v, sem).wait()
        pltpu.sync_copy(rows_v, out_hbm.at[pl.ds(base, B_PER_W)])

    return _sc_gather


CBB = 256


def _combine_body(pm_ref, ps_ref, tci_ref, tcm_ref, out_ref):
    pm = pm_ref[...]
    ps = ps_ref[...]
    lane = lax.broadcasted_iota(jnp.int32, (CBB, 16), 1)
    rowmax = jnp.max(pm, axis=1, keepdims=True)
    col = ps * 16 + lane
    sc_idx = jnp.min(
        jnp.where(pm == rowmax, col, jnp.int32(2**30)),
        axis=1, keepdims=True,
    )
    better_tc = tcm_ref[...] > rowmax
    out_ref[...] = jnp.where(better_tc, tci_ref[...], sc_idx)


_combine_call = pl.pallas_call(
    _combine_body,
    grid=(B // CBB,),
    in_specs=[
        pl.BlockSpec((CBB, 16), lambda i: (i, 0)),
        pl.BlockSpec((CBB, 16), lambda i: (i, 0)),
        pl.BlockSpec((CBB, 1), lambda i: (i, 0)),
        pl.BlockSpec((CBB, 1), lambda i: (i, 0)),
    ],
    out_specs=pl.BlockSpec((CBB, 1), lambda i: (i, 0)),
    out_shape=jax.ShapeDtypeStruct((B, 1), jnp.int32),
)


@jax.jit
def kernel(x, table):
    pm, ps = _make_sc_argmax()(x)
    tci, tcm = _tc_call(x)
    ids = _combine_call(
        pm.reshape(B, 16), ps.reshape(B, 16), tci, tcm
    )
    return _make_sc_gather()(table, ids[:, 0])
